# Initial kernel scaffold; baseline (speedup 1.0000x reference)
import jax
import jax.numpy as jnp
from jax.experimental import pallas as pl


def kernel(x, emb_weight):
    raise NotImplementedError("write your pallas kernel here")


# speedup vs baseline: 1.3285x; 1.3285x over previous
import jax
import jax.numpy as jnp
from jax import lax
from jax.experimental import pallas as pl

EMB = 512
P = 1024
B = 8


def _vq_body(z_ref, w_ref, out_ref):
    z = z_ref[0]
    w = w_ref[...]
    cross = lax.dot_general(z, w, (((0,), (0,)), ((), ())),
                            preferred_element_type=jnp.float32)
    z2 = jnp.sum(z * z, axis=0)
    w2 = jnp.sum(w * w, axis=0)
    dist = (z2[:, None] + w2[None, :]) - 2.0 * cross
    m = jnp.min(dist, axis=1, keepdims=True)
    kio = lax.broadcasted_iota(jnp.int32, (P, EMB), 1)
    idx = jnp.min(jnp.where(dist == m, kio, EMB), axis=1)
    onehot = (kio == idx[:, None]).astype(jnp.float32)
    q = lax.dot_general(w, onehot, (((1,), (1,)), ((), ())),
                        precision=lax.Precision.HIGHEST,
                        preferred_element_type=jnp.float32)
    out_ref[0] = q


def kernel(x, emb_weight):
    z3 = x.reshape(B, EMB, P)
    q3 = pl.pallas_call(
        _vq_body,
        grid=(B,),
        in_specs=[
            pl.BlockSpec((1, EMB, P), lambda b: (b, 0, 0)),
            pl.BlockSpec((EMB, EMB), lambda b: (0, 0)),
        ],
        out_specs=pl.BlockSpec((1, EMB, P), lambda b: (b, 0, 0)),
        out_shape=jax.ShapeDtypeStruct((B, EMB, P), jnp.float32),
    )(z3, emb_weight)
    return q3, x, q3.reshape(x.shape)
